# Initial kernel scaffold; baseline (speedup 1.0000x reference)
import jax
import jax.numpy as jnp
from jax.experimental import pallas as pl


def kernel(x_0, incidence_1, v2e_K, v2e_Q, v2e_V, v2e_ln0_w, v2e_ln0_b, v2e_ln1_w, v2e_ln1_b, v2e_W1, v2e_b1, v2e_W2, v2e_b2, e2v_K, e2v_Q, e2v_V, e2v_ln0_w, e2v_ln0_b, e2v_ln1_w, e2v_ln1_b, e2v_W1, e2v_b1, e2v_W2, e2v_b2):
    raise NotImplementedError("write your pallas kernel here")


# speedup vs baseline: 56.4659x; 56.4659x over previous
import functools

import jax
import jax.numpy as jnp
from jax import lax
from jax.experimental import pallas as pl
from jax.experimental.pallas import tpu as pltpu
from jax.experimental.pallas import tpu_sc as plsc

N_NODES = 10000
N_HE = 10000
N_INC = 320000
D = 128
HEADS = 4
C = D // HEADS
TW = D + 16
ROWS = 1000
NC, NS = 2, 16
NW = NC * NS
EPW = N_INC // NW
CH = 80
CPW = EPW // CH
TACC = 10240
RPT = TACC // NS


def _dot(a, b):
    return lax.dot_general(a, b, (((1,), (0,)), ((), ())),
                           precision=lax.Precision.HIGHEST,
                           preferred_element_type=jnp.float32)


def _pre_math(x, kf, m128, qs16, vf):
    xk = _dot(x, kf)
    es128 = jnp.exp(_dot(xk, m128))
    es16 = jnp.exp(_dot(xk, qs16))
    xv = _dot(x, vf)
    return es128 * xv, es16


def _ln(z, w, b):
    mu = jnp.mean(z, axis=1, keepdims=True)
    var = jnp.mean((z - mu) ** 2, axis=1, keepdims=True)
    return (z - mu) * lax.rsqrt(var + 1e-5) * w + b


def _post_math(acc, g, qf, ln0w, ln0b, ln1w, ln1b, w1, b1, w2, b2):
    num = acc[:, :D]
    den = _dot(acc, g)
    z = jnp.where(den == 0.0, 0.0, num / jnp.where(den == 0.0, 1.0, den)) + qf
    z = _ln(z, ln0w, ln0b)
    h = _dot(jnp.maximum(_dot(z, w1) + b1, 0.0), w2) + b2
    z = z + jnp.maximum(h, 0.0)
    return jnp.maximum(_ln(z, ln1w, ln1b), 0.0)


def _pre_kernel(x_ref, kf_ref, m_ref, qs_ref, vf_ref, exv_ref, es_ref):
    exv, es = _pre_math(x_ref[...], kf_ref[...], m_ref[...], qs_ref[...], vf_ref[...])
    exv_ref[...] = exv
    es_ref[...] = es


def _mid_kernel(p0_ref, p1_ref, g_ref, qf_ref, l0w_ref, l0b_ref, l1w_ref, l1b_ref,
                w1_ref, b1_ref, w2_ref, b2_ref, kf_ref, m_ref, qs_ref, vf_ref,
                x1_ref, exv_ref, es_ref):
    acc = p0_ref[...] + p1_ref[...]
    x1 = _post_math(acc, g_ref[...], qf_ref[...], l0w_ref[...], l0b_ref[...],
                    l1w_ref[...], l1b_ref[...], w1_ref[...], b1_ref[...],
                    w2_ref[...], b2_ref[...])
    x1_ref[...] = x1
    exv, es = _pre_math(x1, kf_ref[...], m_ref[...], qs_ref[...], vf_ref[...])
    exv_ref[...] = exv
    es_ref[...] = es


def _post_kernel(p0_ref, p1_ref, g_ref, qf_ref, l0w_ref, l0b_ref, l1w_ref, l1b_ref,
                 w1_ref, b1_ref, w2_ref, b2_ref, out_ref):
    acc = p0_ref[...] + p1_ref[...]
    out_ref[...] = _post_math(acc, g_ref[...], qf_ref[...], l0w_ref[...], l0b_ref[...],
                              l1w_ref[...], l1b_ref[...], w1_ref[...], b1_ref[...],
                              w2_ref[...], b2_ref[...])


def _row_spec(w):
    return pl.BlockSpec((ROWS, w), lambda i: (i, 0))


def _full_spec(r, w):
    return pl.BlockSpec((r, w), lambda i: (0, 0))


_W_SPECS_POST = [_full_spec(TW, D), _full_spec(1, D), _full_spec(1, D), _full_spec(1, D),
                 _full_spec(1, D), _full_spec(1, D), _full_spec(D, D), _full_spec(1, D),
                 _full_spec(D, D), _full_spec(1, D)]
_W_SPECS_PRE = [_full_spec(D, D), _full_spec(D, D), _full_spec(D, 16), _full_spec(D, D)]


def _run_pre(x, kf, m128, qs16, vf, n):
    return pl.pallas_call(
        _pre_kernel,
        grid=(n // ROWS,),
        in_specs=[_row_spec(D)] + _W_SPECS_PRE,
        out_specs=[_row_spec(D), _row_spec(16)],
        out_shape=[jax.ShapeDtypeStruct((n, D), jnp.float32),
                   jax.ShapeDtypeStruct((n, 16), jnp.float32)],
    )(x, kf, m128, qs16, vf)


def _run_mid(p0, p1, postw, prew, n):
    return pl.pallas_call(
        _mid_kernel,
        grid=(n // ROWS,),
        in_specs=[_row_spec(TW), _row_spec(TW)] + _W_SPECS_POST + _W_SPECS_PRE,
        out_specs=[_row_spec(D), _row_spec(D), _row_spec(16)],
        out_shape=[jax.ShapeDtypeStruct((n, D), jnp.float32),
                   jax.ShapeDtypeStruct((n, D), jnp.float32),
                   jax.ShapeDtypeStruct((n, 16), jnp.float32)],
    )(p0, p1, *postw, *prew)


def _run_post(p0, p1, postw, n):
    return pl.pallas_call(
        _post_kernel,
        grid=(n // ROWS,),
        in_specs=[_row_spec(TW), _row_spec(TW)] + _W_SPECS_POST,
        out_specs=_row_spec(D),
        out_shape=jax.ShapeDtypeStruct((n, D), jnp.float32),
    )(p0, p1, *postw)


def _sc_body(table_hbm, src_hbm, tgt_hbm, zeros_hbm, out_hbm,
             src_v, tgt_v, rows_v, acc_sh):
    cid = lax.axis_index("c")
    sid = lax.axis_index("s")
    wid = cid * NS + sid
    pltpu.sync_copy(zeros_hbm, rows_v)

    def zbody(j, carry):
        row0 = pl.multiple_of(sid * RPT + j * CH, 8)
        pltpu.sync_copy(rows_v, acc_sh.at[pl.ds(row0, CH)])
        return carry

    lax.fori_loop(0, RPT // CH, zbody, 0)
    plsc.subcore_barrier()
    base = pl.multiple_of(wid * EPW, 8)

    def body(k, carry):
        off = pl.multiple_of(base + k * CH, 8)
        pltpu.sync_copy(src_hbm.at[pl.ds(off, CH)], src_v)
        pltpu.sync_copy(tgt_hbm.at[pl.ds(off, CH)], tgt_v)
        pltpu.sync_copy(table_hbm.at[src_v], rows_v)
        pltpu.sync_copy(rows_v, acc_sh.at[tgt_v], add=True)
        return carry

    lax.fori_loop(0, CPW, body, 0)
    plsc.subcore_barrier()

    def dbody(j, carry):
        row0 = pl.multiple_of(sid * RPT + j * CH, 8)
        pltpu.sync_copy(acc_sh.at[pl.ds(row0, CH)], rows_v)
        pltpu.sync_copy(rows_v, out_hbm.at[pl.ds(cid * TACC + row0, CH)])
        return carry

    lax.fori_loop(0, RPT // CH, dbody, 0)


@functools.cache
def _make_sc_segsum():
    return pl.kernel(
        _sc_body,
        out_type=jax.ShapeDtypeStruct((NC * TACC, TW), jnp.float32),
        mesh=plsc.VectorSubcoreMesh(core_axis_name="c", subcore_axis_name="s"),
        compiler_params=pltpu.CompilerParams(use_tc_tiling_on_sc=False),
        scratch_types=[
            pltpu.VMEM((CH,), jnp.int32),
            pltpu.VMEM((CH,), jnp.int32),
            pltpu.VMEM((CH, TW), jnp.float32),
            pltpu.VMEM_SHARED((TACC, TW), jnp.float32),
        ],
    )


def _sc_segsum(table, src, tgt, zeros_blk):
    return _make_sc_segsum()(table, src, tgt, zeros_blk)


def _prep_weights(k, q, v):
    kf = k.transpose(1, 0, 2).reshape(D, D)
    vf = v.transpose(1, 0, 2).reshape(D, D)
    qflat = q[:, 0, :].reshape(1, D)
    m128 = jnp.zeros((D, D), jnp.float32)
    qs16 = jnp.zeros((D, 16), jnp.float32)
    for h in range(HEADS):
        sl = slice(h * C, (h + 1) * C)
        m128 = m128.at[sl, sl].set(jnp.broadcast_to(q[h, 0][:, None], (C, C)))
        qs16 = qs16.at[sl, h].set(q[h, 0])
    return kf, vf, qflat, m128, qs16


def _den_selector():
    g = jnp.zeros((TW, D), jnp.float32)
    for h in range(HEADS):
        g = g.at[D + h, h * C:(h + 1) * C].set(1.0)
    return g


def kernel(x_0, incidence_1,
           v2e_K, v2e_Q, v2e_V, v2e_ln0_w, v2e_ln0_b, v2e_ln1_w, v2e_ln1_b,
           v2e_W1, v2e_b1, v2e_W2, v2e_b2,
           e2v_K, e2v_Q, e2v_V, e2v_ln0_w, e2v_ln0_b, e2v_ln1_w, e2v_ln1_b,
           e2v_W1, e2v_b1, e2v_W2, e2v_b2):
    node_idx = incidence_1[0].astype(jnp.int32)
    he_idx = incidence_1[1].astype(jnp.int32)
    zeros_blk = jnp.zeros((CH, TW), jnp.float32)
    g = _den_selector()

    kf_a, vf_a, qf_a, m_a, qs_a = _prep_weights(v2e_K, v2e_Q, v2e_V)
    kf_b, vf_b, qf_b, m_b, qs_b = _prep_weights(e2v_K, e2v_Q, e2v_V)
    r1 = lambda w: w.reshape(1, D)
    postw_a = [g, qf_a, r1(v2e_ln0_w), r1(v2e_ln0_b), r1(v2e_ln1_w), r1(v2e_ln1_b),
               v2e_W1, r1(v2e_b1), v2e_W2, r1(v2e_b2)]
    postw_b = [g, qf_b, r1(e2v_ln0_w), r1(e2v_ln0_b), r1(e2v_ln1_w), r1(e2v_ln1_b),
               e2v_W1, r1(e2v_b1), e2v_W2, r1(e2v_b2)]
    prew_b = [kf_b, m_b, qs_b, vf_b]

    exv_a, es_a = _run_pre(x_0, kf_a, m_a, qs_a, vf_a, N_NODES)
    table_a = jnp.concatenate([exv_a, es_a], axis=1)
    parts_a = _sc_segsum(table_a, node_idx, he_idx, zeros_blk)
    x_1, exv_b, es_b = _run_mid(parts_a[:N_HE], parts_a[TACC:TACC + N_HE],
                                postw_a, prew_b, N_HE)

    table_b = jnp.concatenate([exv_b, es_b], axis=1)
    parts_b = _sc_segsum(table_b, he_idx, node_idx, zeros_blk)
    x_0_out = _run_post(parts_b[:N_NODES], parts_b[TACC:TACC + N_NODES],
                        postw_b, N_NODES)
    return (x_0_out, x_1)


# speedup vs baseline: 78.6076x; 1.3921x over previous
import functools

import jax
import jax.numpy as jnp
from jax import lax
from jax.experimental import pallas as pl
from jax.experimental.pallas import tpu as pltpu
from jax.experimental.pallas import tpu_sc as plsc

N_NODES = 10000
N_HE = 10000
N_INC = 320000
D = 128
HEADS = 4
C = D // HEADS
TW = D + 16
ROWS = 1000
NC, NS = 2, 16
NW = NC * NS
EPW = N_INC // NW
CH = 80
CPW = EPW // CH
TACC = 10240
RPT = TACC // NS


def _dot(a, b):
    return lax.dot_general(a, b, (((1,), (0,)), ((), ())),
                           precision=lax.Precision.HIGHEST,
                           preferred_element_type=jnp.float32)


def _pre_math(x, kf, m128, qs16, vf):
    xk = _dot(x, kf)
    es128 = jnp.exp(_dot(xk, m128))
    es16 = jnp.exp(_dot(xk, qs16))
    xv = _dot(x, vf)
    return es128 * xv, es16


def _ln(z, w, b):
    mu = jnp.mean(z, axis=1, keepdims=True)
    var = jnp.mean((z - mu) ** 2, axis=1, keepdims=True)
    return (z - mu) * lax.rsqrt(var + 1e-5) * w + b


def _post_math(acc, g, qf, ln0w, ln0b, ln1w, ln1b, w1, b1, w2, b2):
    num = acc[:, :D]
    den = _dot(acc, g)
    z = jnp.where(den == 0.0, 0.0, num / jnp.where(den == 0.0, 1.0, den)) + qf
    z = _ln(z, ln0w, ln0b)
    h = _dot(jnp.maximum(_dot(z, w1) + b1, 0.0), w2) + b2
    z = z + jnp.maximum(h, 0.0)
    return jnp.maximum(_ln(z, ln1w, ln1b), 0.0)


def _pre_kernel(x_ref, kf_ref, m_ref, qs_ref, vf_ref, exv_ref, es_ref):
    exv, es = _pre_math(x_ref[...], kf_ref[...], m_ref[...], qs_ref[...], vf_ref[...])
    exv_ref[...] = exv
    es_ref[...] = es


def _mid_kernel(p0_ref, p1_ref, g_ref, qf_ref, l0w_ref, l0b_ref, l1w_ref, l1b_ref,
                w1_ref, b1_ref, w2_ref, b2_ref, kf_ref, m_ref, qs_ref, vf_ref,
                x1_ref, exv_ref, es_ref):
    acc = p0_ref[...] + p1_ref[...]
    x1 = _post_math(acc, g_ref[...], qf_ref[...], l0w_ref[...], l0b_ref[...],
                    l1w_ref[...], l1b_ref[...], w1_ref[...], b1_ref[...],
                    w2_ref[...], b2_ref[...])
    x1_ref[...] = x1
    exv, es = _pre_math(x1, kf_ref[...], m_ref[...], qs_ref[...], vf_ref[...])
    exv_ref[...] = exv
    es_ref[...] = es


def _post_kernel(p0_ref, p1_ref, g_ref, qf_ref, l0w_ref, l0b_ref, l1w_ref, l1b_ref,
                 w1_ref, b1_ref, w2_ref, b2_ref, out_ref):
    acc = p0_ref[...] + p1_ref[...]
    out_ref[...] = _post_math(acc, g_ref[...], qf_ref[...], l0w_ref[...], l0b_ref[...],
                              l1w_ref[...], l1b_ref[...], w1_ref[...], b1_ref[...],
                              w2_ref[...], b2_ref[...])


def _row_spec(w):
    return pl.BlockSpec((ROWS, w), lambda i: (i, 0))


def _full_spec(r, w):
    return pl.BlockSpec((r, w), lambda i: (0, 0))


_W_SPECS_POST = [_full_spec(TW, D), _full_spec(1, D), _full_spec(1, D), _full_spec(1, D),
                 _full_spec(1, D), _full_spec(1, D), _full_spec(D, D), _full_spec(1, D),
                 _full_spec(D, D), _full_spec(1, D)]
_W_SPECS_PRE = [_full_spec(D, D), _full_spec(D, D), _full_spec(D, 16), _full_spec(D, D)]


def _run_pre(x, kf, m128, qs16, vf, n):
    return pl.pallas_call(
        _pre_kernel,
        grid=(n // ROWS,),
        in_specs=[_row_spec(D)] + _W_SPECS_PRE,
        out_specs=[_row_spec(D), _row_spec(16)],
        out_shape=[jax.ShapeDtypeStruct((n, D), jnp.float32),
                   jax.ShapeDtypeStruct((n, 16), jnp.float32)],
    )(x, kf, m128, qs16, vf)


def _run_mid(p0, p1, postw, prew, n):
    return pl.pallas_call(
        _mid_kernel,
        grid=(n // ROWS,),
        in_specs=[_row_spec(TW), _row_spec(TW)] + _W_SPECS_POST + _W_SPECS_PRE,
        out_specs=[_row_spec(D), _row_spec(D), _row_spec(16)],
        out_shape=[jax.ShapeDtypeStruct((n, D), jnp.float32),
                   jax.ShapeDtypeStruct((n, D), jnp.float32),
                   jax.ShapeDtypeStruct((n, 16), jnp.float32)],
    )(p0, p1, *postw, *prew)


def _run_post(p0, p1, postw, n):
    return pl.pallas_call(
        _post_kernel,
        grid=(n // ROWS,),
        in_specs=[_row_spec(TW), _row_spec(TW)] + _W_SPECS_POST,
        out_specs=_row_spec(D),
        out_shape=jax.ShapeDtypeStruct((n, D), jnp.float32),
    )(p0, p1, *postw)


def _sc_body(table_hbm, src_hbm, tgt_hbm, zeros_hbm, out_hbm,
             src_a, tgt_a, rows_a, src_b, tgt_b, rows_b, sem_a, sem_b, acc_sh):
    cid = lax.axis_index("c")
    sid = lax.axis_index("s")
    wid = cid * NS + sid
    pltpu.sync_copy(zeros_hbm, rows_a)

    def zbody(j, carry):
        row0 = pl.multiple_of(sid * RPT + j * CH, 8)
        pltpu.sync_copy(rows_a, acc_sh.at[pl.ds(row0, CH)])
        return carry

    lax.fori_loop(0, RPT // CH, zbody, 0)
    plsc.subcore_barrier()
    base = pl.multiple_of(wid * EPW, 8)

    def stage_and_fire(k, src_v, tgt_v, rows_v, sem):
        off = pl.multiple_of(base + k * CH, 8)
        pltpu.sync_copy(src_hbm.at[pl.ds(off, CH)], src_v)
        pltpu.sync_copy(tgt_hbm.at[pl.ds(off, CH)], tgt_v)
        pltpu.async_copy(table_hbm.at[src_v], rows_v, sem)

    stage_and_fire(0, src_a, tgt_a, rows_a, sem_a)

    def body(i, carry):
        stage_and_fire(2 * i + 1, src_b, tgt_b, rows_b, sem_b)
        pltpu.make_async_copy(table_hbm.at[src_a], rows_a, sem_a).wait()
        pltpu.sync_copy(rows_a, acc_sh.at[tgt_a], add=True)
        stage_and_fire(2 * i + 2, src_a, tgt_a, rows_a, sem_a)
        pltpu.make_async_copy(table_hbm.at[src_b], rows_b, sem_b).wait()
        pltpu.sync_copy(rows_b, acc_sh.at[tgt_b], add=True)
        return carry

    lax.fori_loop(0, (CPW - 1) // 2, body, 0)
    pltpu.make_async_copy(table_hbm.at[src_a], rows_a, sem_a).wait()
    pltpu.sync_copy(rows_a, acc_sh.at[tgt_a], add=True)
    plsc.subcore_barrier()

    def dbody(j, carry):
        row0 = pl.multiple_of(sid * RPT + j * CH, 8)
        pltpu.sync_copy(acc_sh.at[pl.ds(row0, CH)], rows_a)
        pltpu.sync_copy(rows_a, out_hbm.at[pl.ds(cid * TACC + row0, CH)])
        return carry

    lax.fori_loop(0, RPT // CH, dbody, 0)


@functools.cache
def _make_sc_segsum():
    return pl.kernel(
        _sc_body,
        out_type=jax.ShapeDtypeStruct((NC * TACC, TW), jnp.float32),
        mesh=plsc.VectorSubcoreMesh(core_axis_name="c", subcore_axis_name="s"),
        compiler_params=pltpu.CompilerParams(use_tc_tiling_on_sc=False),
        scratch_types=[
            pltpu.VMEM((CH,), jnp.int32),
            pltpu.VMEM((CH,), jnp.int32),
            pltpu.VMEM((CH, TW), jnp.float32),
            pltpu.VMEM((CH,), jnp.int32),
            pltpu.VMEM((CH,), jnp.int32),
            pltpu.VMEM((CH, TW), jnp.float32),
            pltpu.SemaphoreType.DMA,
            pltpu.SemaphoreType.DMA,
            pltpu.VMEM_SHARED((TACC, TW), jnp.float32),
        ],
    )


def _sc_segsum(table, src, tgt, zeros_blk):
    return _make_sc_segsum()(table, src, tgt, zeros_blk)


def _prep_weights(k, q, v):
    kf = k.transpose(1, 0, 2).reshape(D, D)
    vf = v.transpose(1, 0, 2).reshape(D, D)
    qflat = q[:, 0, :].reshape(1, D)
    m128 = jnp.zeros((D, D), jnp.float32)
    qs16 = jnp.zeros((D, 16), jnp.float32)
    for h in range(HEADS):
        sl = slice(h * C, (h + 1) * C)
        m128 = m128.at[sl, sl].set(jnp.broadcast_to(q[h, 0][:, None], (C, C)))
        qs16 = qs16.at[sl, h].set(q[h, 0])
    return kf, vf, qflat, m128, qs16


def _den_selector():
    g = jnp.zeros((TW, D), jnp.float32)
    for h in range(HEADS):
        g = g.at[D + h, h * C:(h + 1) * C].set(1.0)
    return g


def kernel(x_0, incidence_1,
           v2e_K, v2e_Q, v2e_V, v2e_ln0_w, v2e_ln0_b, v2e_ln1_w, v2e_ln1_b,
           v2e_W1, v2e_b1, v2e_W2, v2e_b2,
           e2v_K, e2v_Q, e2v_V, e2v_ln0_w, e2v_ln0_b, e2v_ln1_w, e2v_ln1_b,
           e2v_W1, e2v_b1, e2v_W2, e2v_b2):
    node_idx = incidence_1[0].astype(jnp.int32)
    he_idx = incidence_1[1].astype(jnp.int32)
    zeros_blk = jnp.zeros((CH, TW), jnp.float32)
    g = _den_selector()

    kf_a, vf_a, qf_a, m_a, qs_a = _prep_weights(v2e_K, v2e_Q, v2e_V)
    kf_b, vf_b, qf_b, m_b, qs_b = _prep_weights(e2v_K, e2v_Q, e2v_V)
    r1 = lambda w: w.reshape(1, D)
    postw_a = [g, qf_a, r1(v2e_ln0_w), r1(v2e_ln0_b), r1(v2e_ln1_w), r1(v2e_ln1_b),
               v2e_W1, r1(v2e_b1), v2e_W2, r1(v2e_b2)]
    postw_b = [g, qf_b, r1(e2v_ln0_w), r1(e2v_ln0_b), r1(e2v_ln1_w), r1(e2v_ln1_b),
               e2v_W1, r1(e2v_b1), e2v_W2, r1(e2v_b2)]
    prew_b = [kf_b, m_b, qs_b, vf_b]

    exv_a, es_a = _run_pre(x_0, kf_a, m_a, qs_a, vf_a, N_NODES)
    table_a = jnp.concatenate([exv_a, es_a], axis=1)
    parts_a = _sc_segsum(table_a, node_idx, he_idx, zeros_blk)
    x_1, exv_b, es_b = _run_mid(parts_a[:N_HE], parts_a[TACC:TACC + N_HE],
                                postw_a, prew_b, N_HE)

    table_b = jnp.concatenate([exv_b, es_b], axis=1)
    parts_b = _sc_segsum(table_b, he_idx, node_idx, zeros_blk)
    x_0_out = _run_post(parts_b[:N_NODES], parts_b[TACC:TACC + N_NODES],
                        postw_b, N_NODES)
    return (x_0_out, x_1)
